# Initial kernel scaffold; baseline (speedup 1.0000x reference)
import jax
import jax.numpy as jnp
from jax.experimental import pallas as pl


def kernel(x, edge_index, edge_attr, edge_t, W_m, b_m, W_r, b_r):
    raise NotImplementedError("write your pallas kernel here")


# speedup vs baseline: 3.5326x; 3.5326x over previous
import functools
import jax
import jax.numpy as jnp
from jax import lax
from jax.experimental import pallas as pl
from jax.experimental.pallas import tpu as pltpu
from jax.experimental.pallas import tpu_sc as plsc

N_NODES = 10000
NODE_DIM = 128
EDGE_DIM = 16
OUT_DIM = 128
N_EDGES = 320000

NC = 2
NS = 16
HALF = OUT_DIM // NC
E_PER_S = N_EDGES // NS
CH = 128
NFULL = E_PER_S // CH
TAIL = E_PER_S - NFULL * CH
CP_ROWS = 632
CP_LAST = N_NODES - (NS - 1) * CP_ROWS

_ROW_BLK = 1000
_GRID = N_NODES // _ROW_BLK


def _tc1_body(x_ref, wxs_ref, wrt_ref, bias_ref, xw1_ref, xw2_ref, base_ref):
    x = x_ref[...]
    xw = jnp.dot(x, wxs_ref[...], preferred_element_type=jnp.float32)
    xw1_ref[...] = xw[:, :HALF]
    xw2_ref[...] = xw[:, HALF:]
    base_ref[...] = (
        jnp.dot(x, wrt_ref[...], preferred_element_type=jnp.float32)
        + bias_ref[...]
    )


def _tc1(x, wxs, wrt, bias):
    return pl.pallas_call(
        _tc1_body,
        grid=(_GRID,),
        in_specs=[
            pl.BlockSpec((_ROW_BLK, NODE_DIM), lambda i: (i, 0)),
            pl.BlockSpec((NODE_DIM, OUT_DIM), lambda i: (0, 0)),
            pl.BlockSpec((NODE_DIM, OUT_DIM), lambda i: (0, 0)),
            pl.BlockSpec((1, OUT_DIM), lambda i: (0, 0)),
        ],
        out_specs=[
            pl.BlockSpec((_ROW_BLK, HALF), lambda i: (i, 0)),
            pl.BlockSpec((_ROW_BLK, HALF), lambda i: (i, 0)),
            pl.BlockSpec((_ROW_BLK, OUT_DIM), lambda i: (i, 0)),
        ],
        out_shape=[
            jax.ShapeDtypeStruct((N_NODES, HALF), jnp.float32),
            jax.ShapeDtypeStruct((N_NODES, HALF), jnp.float32),
            jax.ShapeDtypeStruct((N_NODES, OUT_DIM), jnp.float32),
        ],
    )(x, wxs, wrt, bias)


def _sc_body(row_hbm, col_hbm, ea_hbm, et_hbm, xw1_hbm, xw2_hbm,
             z64_hbm, z16_hbm,
             a1_out, a2_out, a3_out,
             row_v, col_v, rows_v, ea_v,
             row_t, col_t, rows_t, ea_t,
             a1sh, aesh, sem):
    c = lax.axis_index("c")
    s = lax.axis_index("s")
    eb = s * E_PER_S
    r0 = pl.multiple_of(s * CP_ROWS, 8)

    @pl.when(s < NS - 1)
    def _zero_main():
        pltpu.sync_copy(z64_hbm, a1sh.at[pl.ds(r0, CP_ROWS)])
        pltpu.sync_copy(z16_hbm, aesh.at[pl.ds(r0, CP_ROWS)])

    @pl.when(s == NS - 1)
    def _zero_last():
        q0 = (NS - 1) * CP_ROWS
        pltpu.sync_copy(z64_hbm.at[pl.ds(0, CP_LAST)],
                        a1sh.at[pl.ds(q0, CP_LAST)])
        pltpu.sync_copy(z16_hbm.at[pl.ds(0, CP_LAST)],
                        aesh.at[pl.ds(q0, CP_LAST)])

    plsc.subcore_barrier()

    def run_side(xw_hbm, eat_hbm):
        def chunk(i, _):
            off = pl.multiple_of(eb + i * CH, CH)
            pltpu.sync_copy(row_hbm.at[pl.ds(off, CH)], row_v)
            pltpu.sync_copy(col_hbm.at[pl.ds(off, CH)], col_v)
            cp = pltpu.async_copy(xw_hbm.at[col_v], rows_v, sem)
            pltpu.sync_copy(eat_hbm.at[pl.ds(off, CH)], ea_v)
            cp.wait()
            pltpu.sync_copy(rows_v, a1sh.at[row_v], add=True)
            pltpu.sync_copy(ea_v, aesh.at[row_v], add=True)
            return 0
        lax.fori_loop(0, NFULL, chunk, 0)

        toff = pl.multiple_of(eb + NFULL * CH, 16)
        pltpu.sync_copy(row_hbm.at[pl.ds(toff, TAIL)], row_t)
        pltpu.sync_copy(col_hbm.at[pl.ds(toff, TAIL)], col_t)
        cp = pltpu.async_copy(xw_hbm.at[col_t], rows_t, sem)
        pltpu.sync_copy(eat_hbm.at[pl.ds(toff, TAIL)], ea_t)
        cp.wait()
        pltpu.sync_copy(rows_t, a1sh.at[row_t], add=True)
        pltpu.sync_copy(ea_t, aesh.at[row_t], add=True)

    @pl.when(c == 0)
    def _side0():
        run_side(xw1_hbm, ea_hbm)

    @pl.when(c == 1)
    def _side1():
        run_side(xw2_hbm, et_hbm)

    plsc.subcore_barrier()

    @pl.when(s < NS - 1)
    def _out_main():
        pltpu.sync_copy(a1sh.at[pl.ds(r0, CP_ROWS)],
                        a1_out.at[c, pl.ds(r0, CP_ROWS)])

    @pl.when(s == NS - 1)
    def _out_last():
        q0 = (NS - 1) * CP_ROWS
        pltpu.sync_copy(a1sh.at[pl.ds(q0, CP_LAST)],
                        a1_out.at[c, pl.ds(q0, CP_LAST)])

    @pl.when(jnp.logical_and(c == 0, s < NS - 1))
    def _ae_main0():
        pltpu.sync_copy(aesh.at[pl.ds(r0, CP_ROWS)],
                        a2_out.at[pl.ds(r0, CP_ROWS)])

    @pl.when(jnp.logical_and(c == 0, s == NS - 1))
    def _ae_last0():
        q0 = (NS - 1) * CP_ROWS
        pltpu.sync_copy(aesh.at[pl.ds(q0, CP_LAST)],
                        a2_out.at[pl.ds(q0, CP_LAST)])

    @pl.when(jnp.logical_and(c == 1, s < NS - 1))
    def _ae_main1():
        pltpu.sync_copy(aesh.at[pl.ds(r0, CP_ROWS)],
                        a3_out.at[pl.ds(r0, CP_ROWS)])

    @pl.when(jnp.logical_and(c == 1, s == NS - 1))
    def _ae_last1():
        q0 = (NS - 1) * CP_ROWS
        pltpu.sync_copy(aesh.at[pl.ds(q0, CP_LAST)],
                        a3_out.at[pl.ds(q0, CP_LAST)])


_sc_agg = functools.partial(
    pl.kernel,
    out_type=[
        jax.ShapeDtypeStruct((NC, N_NODES, HALF), jnp.float32),
        jax.ShapeDtypeStruct((N_NODES, EDGE_DIM), jnp.float32),
        jax.ShapeDtypeStruct((N_NODES, EDGE_DIM), jnp.float32),
    ],
    mesh=plsc.VectorSubcoreMesh(core_axis_name="c", subcore_axis_name="s"),
    compiler_params=pltpu.CompilerParams(use_tc_tiling_on_sc=False),
    scratch_types=[
        pltpu.VMEM((CH,), jnp.int32),
        pltpu.VMEM((CH,), jnp.int32),
        pltpu.VMEM((CH, HALF), jnp.float32),
        pltpu.VMEM((CH, EDGE_DIM), jnp.float32),
        pltpu.VMEM((TAIL,), jnp.int32),
        pltpu.VMEM((TAIL,), jnp.int32),
        pltpu.VMEM((TAIL, HALF), jnp.float32),
        pltpu.VMEM((TAIL, EDGE_DIM), jnp.float32),
        pltpu.VMEM_SHARED((N_NODES, HALF), jnp.float32),
        pltpu.VMEM_SHARED((N_NODES, EDGE_DIM), jnp.float32),
        pltpu.SemaphoreType.DMA,
    ],
)(_sc_body)


def _tc2_body(a1_ref, a2_ref, a3_ref, was_ref, wts_ref, base_ref, out_ref):
    a1 = jnp.concatenate([a1_ref[0], a1_ref[1]], axis=-1)
    out_ref[...] = (
        a1 + base_ref[...]
        + jnp.dot(a2_ref[...], was_ref[...], preferred_element_type=jnp.float32)
        + jnp.dot(a3_ref[...], wts_ref[...], preferred_element_type=jnp.float32)
    )


def _tc2(a1, a2, a3, was, wts, base):
    return pl.pallas_call(
        _tc2_body,
        grid=(_GRID,),
        in_specs=[
            pl.BlockSpec((NC, _ROW_BLK, HALF), lambda i: (0, i, 0)),
            pl.BlockSpec((_ROW_BLK, EDGE_DIM), lambda i: (i, 0)),
            pl.BlockSpec((_ROW_BLK, EDGE_DIM), lambda i: (i, 0)),
            pl.BlockSpec((EDGE_DIM, OUT_DIM), lambda i: (0, 0)),
            pl.BlockSpec((EDGE_DIM, OUT_DIM), lambda i: (0, 0)),
            pl.BlockSpec((_ROW_BLK, OUT_DIM), lambda i: (i, 0)),
        ],
        out_specs=pl.BlockSpec((_ROW_BLK, OUT_DIM), lambda i: (i, 0)),
        out_shape=jax.ShapeDtypeStruct((N_NODES, OUT_DIM), jnp.float32),
    )(a1, a2, a3, was, wts, base)


def kernel(x, edge_index, edge_attr, edge_t, W_m, b_m, W_r, b_r):
    ei = edge_index.astype(jnp.int32)
    row = ei[0]
    col = ei[1]
    wxs = 0.5 * W_m[:, :NODE_DIM].T
    was = 0.5 * W_m[:, NODE_DIM:NODE_DIM + EDGE_DIM].T
    wts = 0.5 * W_m[:, NODE_DIM + EDGE_DIM:].T
    bias = (b_r + 0.5 * b_m).reshape(1, OUT_DIM)
    z64 = jnp.zeros((CP_ROWS, HALF), jnp.float32)
    z16 = jnp.zeros((CP_ROWS, EDGE_DIM), jnp.float32)

    xw1, xw2, base = _tc1(x, wxs, W_r.T, bias)
    a1, a2, a3 = _sc_agg(row, col, edge_attr, edge_t, xw1, xw2, z64, z16)
    return _tc2(a1, a2, a3, was, wts, base)


# speedup vs baseline: 3.8722x; 1.0961x over previous
import functools
import jax
import jax.numpy as jnp
from jax import lax
from jax.experimental import pallas as pl
from jax.experimental.pallas import tpu as pltpu
from jax.experimental.pallas import tpu_sc as plsc

N_NODES = 10000
NODE_DIM = 128
EDGE_DIM = 16
OUT_DIM = 128
N_EDGES = 320000

NC = 2
NS = 16
HALF = OUT_DIM // NC
BLK = 128
NBLOCKS = N_EDGES // BLK
BPT = 156
K_MAIN = BPT // 4
K_LAST = (NBLOCKS - (NS - 1) * BPT) // 4
CP_ROWS = 632
CP_LAST = N_NODES - (NS - 1) * CP_ROWS

_ROW_BLK = 1000
_GRID = N_NODES // _ROW_BLK


def _tc1_body(x_ref, wxs_ref, wrt_ref, bias_ref, xw1_ref, xw2_ref, base_ref):
    x = x_ref[...]
    xw = jnp.dot(x, wxs_ref[...], preferred_element_type=jnp.float32)
    xw1_ref[...] = xw[:, :HALF]
    xw2_ref[...] = xw[:, HALF:]
    base_ref[...] = (
        jnp.dot(x, wrt_ref[...], preferred_element_type=jnp.float32)
        + bias_ref[...]
    )


def _tc1(x, wxs, wrt, bias):
    return pl.pallas_call(
        _tc1_body,
        grid=(_GRID,),
        in_specs=[
            pl.BlockSpec((_ROW_BLK, NODE_DIM), lambda i: (i, 0)),
            pl.BlockSpec((NODE_DIM, OUT_DIM), lambda i: (0, 0)),
            pl.BlockSpec((NODE_DIM, OUT_DIM), lambda i: (0, 0)),
            pl.BlockSpec((1, OUT_DIM), lambda i: (0, 0)),
        ],
        out_specs=[
            pl.BlockSpec((_ROW_BLK, HALF), lambda i: (i, 0)),
            pl.BlockSpec((_ROW_BLK, HALF), lambda i: (i, 0)),
            pl.BlockSpec((_ROW_BLK, OUT_DIM), lambda i: (i, 0)),
        ],
        out_shape=[
            jax.ShapeDtypeStruct((N_NODES, HALF), jnp.float32),
            jax.ShapeDtypeStruct((N_NODES, HALF), jnp.float32),
            jax.ShapeDtypeStruct((N_NODES, OUT_DIM), jnp.float32),
        ],
    )(x, wxs, wrt, bias)


def _sc_body(row_hbm, col_hbm, ea_hbm, et_hbm, xw1_hbm, xw2_hbm,
             z64_hbm, z16_hbm,
             a1_out, a2_out, a3_out,
             row00, row01, row10, row11,
             col00, col01, col10, col11,
             rv00, rv01, rv10, rv11,
             ev00, ev01, ev10, ev11,
             a1sh, aesh, sem_g0, sem_g1, sem_s0, sem_s1):
    c = lax.axis_index("c")
    s = lax.axis_index("s")
    r0 = pl.multiple_of(s * CP_ROWS, 8)
    rowb = ((row00, row01), (row10, row11))
    colb = ((col00, col01), (col10, col11))
    rvb = ((rv00, rv01), (rv10, rv11))
    evb = ((ev00, ev01), (ev10, ev11))
    semg = (sem_g0, sem_g1)
    sems = (sem_s0, sem_s1)

    @pl.when(s < NS - 1)
    def _zero_main():
        pltpu.sync_copy(z64_hbm, a1sh.at[pl.ds(r0, CP_ROWS)])
        pltpu.sync_copy(z16_hbm, aesh.at[pl.ds(r0, CP_ROWS)])

    @pl.when(s == NS - 1)
    def _zero_last():
        q0 = (NS - 1) * CP_ROWS
        pltpu.sync_copy(z64_hbm.at[pl.ds(0, CP_LAST)],
                        a1sh.at[pl.ds(q0, CP_LAST)])
        pltpu.sync_copy(z16_hbm.at[pl.ds(0, CP_LAST)],
                        aesh.at[pl.ds(q0, CP_LAST)])

    plsc.subcore_barrier()

    base_blk = s * BPT
    nk = jnp.where(s == NS - 1, K_LAST, K_MAIN)

    def run_side(xw_hbm, eat_hbm):
        def loads(b, blk):
            for j in (0, 1):
                off = pl.multiple_of((blk + j) * BLK, BLK)
                pltpu.sync_copy(row_hbm.at[pl.ds(off, BLK)], rowb[b][j])
                pltpu.sync_copy(col_hbm.at[pl.ds(off, BLK)], colb[b][j])
                pltpu.sync_copy(eat_hbm.at[pl.ds(off, BLK)], evb[b][j])

        def issue_g(b):
            for j in (0, 1):
                pltpu.async_copy(xw_hbm.at[colb[b][j]], rvb[b][j], semg[b])

        def drain_g(b):
            for j in (0, 1):
                pltpu.make_async_copy(
                    xw_hbm.at[colb[b][j]], rvb[b][j], semg[b]).wait()

        def issue_s(b):
            for j in (0, 1):
                pltpu.async_copy(rvb[b][j], a1sh.at[rowb[b][j]],
                                 sems[b], add=True)
                pltpu.async_copy(evb[b][j], aesh.at[rowb[b][j]],
                                 sems[b], add=True)

        def drain_s(b):
            for j in (0, 1):
                pltpu.make_async_copy(
                    rvb[b][j], a1sh.at[rowb[b][j]], sems[b]).wait()
                pltpu.make_async_copy(
                    evb[b][j], aesh.at[rowb[b][j]], sems[b]).wait()

        def body(k, _):
            blk0 = base_blk + k * 4
            pl.when(k > 0)(lambda: drain_s(0))
            loads(0, blk0)
            issue_g(0)

            def _mid():
                drain_g(1)
                issue_s(1)
            pl.when(k > 0)(_mid)
            pl.when(k > 0)(lambda: drain_s(1))
            loads(1, blk0 + 2)
            issue_g(1)
            drain_g(0)
            issue_s(0)
            return 0

        lax.fori_loop(0, nk, body, 0)
        drain_g(1)
        issue_s(1)
        drain_s(0)
        drain_s(1)

    @pl.when(c == 0)
    def _side0():
        run_side(xw1_hbm, ea_hbm)

    @pl.when(c == 1)
    def _side1():
        run_side(xw2_hbm, et_hbm)

    plsc.subcore_barrier()

    @pl.when(s < NS - 1)
    def _out_main():
        pltpu.sync_copy(a1sh.at[pl.ds(r0, CP_ROWS)],
                        a1_out.at[c, pl.ds(r0, CP_ROWS)])

    @pl.when(s == NS - 1)
    def _out_last():
        q0 = (NS - 1) * CP_ROWS
        pltpu.sync_copy(a1sh.at[pl.ds(q0, CP_LAST)],
                        a1_out.at[c, pl.ds(q0, CP_LAST)])

    @pl.when(jnp.logical_and(c == 0, s < NS - 1))
    def _ae_main0():
        pltpu.sync_copy(aesh.at[pl.ds(r0, CP_ROWS)],
                        a2_out.at[pl.ds(r0, CP_ROWS)])

    @pl.when(jnp.logical_and(c == 0, s == NS - 1))
    def _ae_last0():
        q0 = (NS - 1) * CP_ROWS
        pltpu.sync_copy(aesh.at[pl.ds(q0, CP_LAST)],
                        a2_out.at[pl.ds(q0, CP_LAST)])

    @pl.when(jnp.logical_and(c == 1, s < NS - 1))
    def _ae_main1():
        pltpu.sync_copy(aesh.at[pl.ds(r0, CP_ROWS)],
                        a3_out.at[pl.ds(r0, CP_ROWS)])

    @pl.when(jnp.logical_and(c == 1, s == NS - 1))
    def _ae_last1():
        q0 = (NS - 1) * CP_ROWS
        pltpu.sync_copy(aesh.at[pl.ds(q0, CP_LAST)],
                        a3_out.at[pl.ds(q0, CP_LAST)])


_sc_agg = functools.partial(
    pl.kernel,
    out_type=[
        jax.ShapeDtypeStruct((NC, N_NODES, HALF), jnp.float32),
        jax.ShapeDtypeStruct((N_NODES, EDGE_DIM), jnp.float32),
        jax.ShapeDtypeStruct((N_NODES, EDGE_DIM), jnp.float32),
    ],
    mesh=plsc.VectorSubcoreMesh(core_axis_name="c", subcore_axis_name="s"),
    compiler_params=pltpu.CompilerParams(use_tc_tiling_on_sc=False),
    scratch_types=(
        [pltpu.VMEM((BLK,), jnp.int32)] * 8
        + [pltpu.VMEM((BLK, HALF), jnp.float32)] * 4
        + [pltpu.VMEM((BLK, EDGE_DIM), jnp.float32)] * 4
        + [
            pltpu.VMEM_SHARED((N_NODES, HALF), jnp.float32),
            pltpu.VMEM_SHARED((N_NODES, EDGE_DIM), jnp.float32),
            pltpu.SemaphoreType.DMA,
            pltpu.SemaphoreType.DMA,
            pltpu.SemaphoreType.DMA,
            pltpu.SemaphoreType.DMA,
        ]
    ),
)(_sc_body)


def _tc2_body(a1_ref, a2_ref, a3_ref, was_ref, wts_ref, base_ref, out_ref):
    a1 = jnp.concatenate([a1_ref[0], a1_ref[1]], axis=-1)
    out_ref[...] = (
        a1 + base_ref[...]
        + jnp.dot(a2_ref[...], was_ref[...], preferred_element_type=jnp.float32)
        + jnp.dot(a3_ref[...], wts_ref[...], preferred_element_type=jnp.float32)
    )


def _tc2(a1, a2, a3, was, wts, base):
    return pl.pallas_call(
        _tc2_body,
        grid=(_GRID,),
        in_specs=[
            pl.BlockSpec((NC, _ROW_BLK, HALF), lambda i: (0, i, 0)),
            pl.BlockSpec((_ROW_BLK, EDGE_DIM), lambda i: (i, 0)),
            pl.BlockSpec((_ROW_BLK, EDGE_DIM), lambda i: (i, 0)),
            pl.BlockSpec((EDGE_DIM, OUT_DIM), lambda i: (0, 0)),
            pl.BlockSpec((EDGE_DIM, OUT_DIM), lambda i: (0, 0)),
            pl.BlockSpec((_ROW_BLK, OUT_DIM), lambda i: (i, 0)),
        ],
        out_specs=pl.BlockSpec((_ROW_BLK, OUT_DIM), lambda i: (i, 0)),
        out_shape=jax.ShapeDtypeStruct((N_NODES, OUT_DIM), jnp.float32),
    )(a1, a2, a3, was, wts, base)


def kernel(x, edge_index, edge_attr, edge_t, W_m, b_m, W_r, b_r):
    ei = edge_index.astype(jnp.int32)
    row = ei[0]
    col = ei[1]
    wxs = 0.5 * W_m[:, :NODE_DIM].T
    was = 0.5 * W_m[:, NODE_DIM:NODE_DIM + EDGE_DIM].T
    wts = 0.5 * W_m[:, NODE_DIM + EDGE_DIM:].T
    bias = (b_r + 0.5 * b_m).reshape(1, OUT_DIM)
    z64 = jnp.zeros((CP_ROWS, HALF), jnp.float32)
    z16 = jnp.zeros((CP_ROWS, EDGE_DIM), jnp.float32)

    xw1, xw2, base = _tc1(x, wxs, W_r.T, bias)
    a1, a2, a3 = _sc_agg(row, col, edge_attr, edge_t, xw1, xw2, z64, z16)
    return _tc2(a1, a2, a3, was, wts, base)


# speedup vs baseline: 3.9249x; 1.0136x over previous
import functools
import jax
import jax.numpy as jnp
from jax import lax
from jax.experimental import pallas as pl
from jax.experimental.pallas import tpu as pltpu
from jax.experimental.pallas import tpu_sc as plsc

N_NODES = 10000
NODE_DIM = 128
EDGE_DIM = 16
OUT_DIM = 128
N_EDGES = 320000

NC = 2
NS = 16
HALF = OUT_DIM // NC
BLK = 128
NBLOCKS = N_EDGES // BLK
BPT = 156
K_MAIN = BPT // 4
K_LAST = (NBLOCKS - (NS - 1) * BPT) // 4
CP_ROWS = 632
CP_LAST = N_NODES - (NS - 1) * CP_ROWS

_ROW_BLK = 1000
_GRID = N_NODES // _ROW_BLK


def _tc1_body(x_ref, wxs_ref, wrt_ref, bias_ref, xw1_ref, xw2_ref, base_ref):
    x = x_ref[...]
    xw = jnp.dot(x, wxs_ref[...], preferred_element_type=jnp.float32)
    xw1_ref[...] = xw[:, :HALF]
    xw2_ref[...] = xw[:, HALF:]
    base_ref[...] = (
        jnp.dot(x, wrt_ref[...], preferred_element_type=jnp.float32)
        + bias_ref[...]
    )


def _tc1(x, wxs, wrt, bias):
    return pl.pallas_call(
        _tc1_body,
        grid=(_GRID,),
        in_specs=[
            pl.BlockSpec((_ROW_BLK, NODE_DIM), lambda i: (i, 0)),
            pl.BlockSpec((NODE_DIM, OUT_DIM), lambda i: (0, 0)),
            pl.BlockSpec((NODE_DIM, OUT_DIM), lambda i: (0, 0)),
            pl.BlockSpec((1, OUT_DIM), lambda i: (0, 0)),
        ],
        out_specs=[
            pl.BlockSpec((_ROW_BLK, HALF), lambda i: (i, 0)),
            pl.BlockSpec((_ROW_BLK, HALF), lambda i: (i, 0)),
            pl.BlockSpec((_ROW_BLK, OUT_DIM), lambda i: (i, 0)),
        ],
        out_shape=[
            jax.ShapeDtypeStruct((N_NODES, HALF), jnp.float32),
            jax.ShapeDtypeStruct((N_NODES, HALF), jnp.float32),
            jax.ShapeDtypeStruct((N_NODES, OUT_DIM), jnp.float32),
        ],
    )(x, wxs, wrt, bias)


def _sc_body(ei_hbm, ea_hbm, et_hbm, xw1_hbm, xw2_hbm,
             z64_hbm, z16_hbm,
             a1c0_out, a1c1_out, a2_out, a3_out,
             row00, row01, row10, row11,
             col00, col01, col10, col11,
             rv00, rv01, rv10, rv11,
             ev00, ev01, ev10, ev11,
             a1sh, aesh, sem_g0, sem_g1, sem_s0, sem_s1):
    c = lax.axis_index("c")
    s = lax.axis_index("s")
    r0 = pl.multiple_of(s * CP_ROWS, 8)
    row_hbm = ei_hbm.at[0]
    col_hbm = ei_hbm.at[1]
    rowb = ((row00, row01), (row10, row11))
    colb = ((col00, col01), (col10, col11))
    rvb = ((rv00, rv01), (rv10, rv11))
    evb = ((ev00, ev01), (ev10, ev11))
    semg = (sem_g0, sem_g1)
    sems = (sem_s0, sem_s1)

    @pl.when(s < NS - 1)
    def _zero_main():
        pltpu.sync_copy(z64_hbm, a1sh.at[pl.ds(r0, CP_ROWS)])
        pltpu.sync_copy(z16_hbm, aesh.at[pl.ds(r0, CP_ROWS)])

    @pl.when(s == NS - 1)
    def _zero_last():
        q0 = (NS - 1) * CP_ROWS
        pltpu.sync_copy(z64_hbm.at[pl.ds(0, CP_LAST)],
                        a1sh.at[pl.ds(q0, CP_LAST)])
        pltpu.sync_copy(z16_hbm.at[pl.ds(0, CP_LAST)],
                        aesh.at[pl.ds(q0, CP_LAST)])

    plsc.subcore_barrier()

    base_blk = s * BPT
    nk = jnp.where(s == NS - 1, K_LAST, K_MAIN)

    def run_side(xw_hbm, eat_hbm):
        def loads(b, blk):
            for j in (0, 1):
                off = pl.multiple_of((blk + j) * BLK, BLK)
                pltpu.sync_copy(row_hbm.at[pl.ds(off, BLK)], rowb[b][j])
                pltpu.sync_copy(col_hbm.at[pl.ds(off, BLK)], colb[b][j])
                pltpu.sync_copy(eat_hbm.at[pl.ds(off, BLK)], evb[b][j])

        def issue_g(b):
            for j in (0, 1):
                pltpu.async_copy(xw_hbm.at[colb[b][j]], rvb[b][j], semg[b])

        def drain_g(b):
            for j in (0, 1):
                pltpu.make_async_copy(
                    xw_hbm.at[colb[b][j]], rvb[b][j], semg[b]).wait()

        def issue_s(b):
            for j in (0, 1):
                pltpu.async_copy(rvb[b][j], a1sh.at[rowb[b][j]],
                                 sems[b], add=True)
                pltpu.async_copy(evb[b][j], aesh.at[rowb[b][j]],
                                 sems[b], add=True)

        def drain_s(b):
            for j in (0, 1):
                pltpu.make_async_copy(
                    rvb[b][j], a1sh.at[rowb[b][j]], sems[b]).wait()
                pltpu.make_async_copy(
                    evb[b][j], aesh.at[rowb[b][j]], sems[b]).wait()

        def body(k, _):
            blk0 = base_blk + k * 4
            pl.when(k > 0)(lambda: drain_s(0))
            loads(0, blk0)
            issue_g(0)

            def _mid():
                drain_g(1)
                issue_s(1)
            pl.when(k > 0)(_mid)
            pl.when(k > 0)(lambda: drain_s(1))
            loads(1, blk0 + 2)
            issue_g(1)
            drain_g(0)
            issue_s(0)
            return 0

        lax.fori_loop(0, nk, body, 0)
        drain_g(1)
        issue_s(1)
        drain_s(0)
        drain_s(1)

    @pl.when(c == 0)
    def _side0():
        run_side(xw1_hbm, ea_hbm)

    @pl.when(c == 1)
    def _side1():
        run_side(xw2_hbm, et_hbm)

    plsc.subcore_barrier()

    def copy_out(a1o, aeo):
        @pl.when(s < NS - 1)
        def _main():
            pltpu.sync_copy(a1sh.at[pl.ds(r0, CP_ROWS)],
                            a1o.at[pl.ds(r0, CP_ROWS)])
            pltpu.sync_copy(aesh.at[pl.ds(r0, CP_ROWS)],
                            aeo.at[pl.ds(r0, CP_ROWS)])

        @pl.when(s == NS - 1)
        def _last():
            q0 = (NS - 1) * CP_ROWS
            pltpu.sync_copy(a1sh.at[pl.ds(q0, CP_LAST)],
                            a1o.at[pl.ds(q0, CP_LAST)])
            pltpu.sync_copy(aesh.at[pl.ds(q0, CP_LAST)],
                            aeo.at[pl.ds(q0, CP_LAST)])

    @pl.when(c == 0)
    def _out0():
        copy_out(a1c0_out, a2_out)

    @pl.when(c == 1)
    def _out1():
        copy_out(a1c1_out, a3_out)


_sc_agg = functools.partial(
    pl.kernel,
    out_type=[
        jax.ShapeDtypeStruct((N_NODES, HALF), jnp.float32),
        jax.ShapeDtypeStruct((N_NODES, HALF), jnp.float32),
        jax.ShapeDtypeStruct((N_NODES, EDGE_DIM), jnp.float32),
        jax.ShapeDtypeStruct((N_NODES, EDGE_DIM), jnp.float32),
    ],
    mesh=plsc.VectorSubcoreMesh(core_axis_name="c", subcore_axis_name="s"),
    compiler_params=pltpu.CompilerParams(use_tc_tiling_on_sc=False),
    scratch_types=(
        [pltpu.VMEM((BLK,), jnp.int32)] * 8
        + [pltpu.VMEM((BLK, HALF), jnp.float32)] * 4
        + [pltpu.VMEM((BLK, EDGE_DIM), jnp.float32)] * 4
        + [
            pltpu.VMEM_SHARED((N_NODES, HALF), jnp.float32),
            pltpu.VMEM_SHARED((N_NODES, EDGE_DIM), jnp.float32),
            pltpu.SemaphoreType.DMA,
            pltpu.SemaphoreType.DMA,
            pltpu.SemaphoreType.DMA,
            pltpu.SemaphoreType.DMA,
        ]
    ),
)(_sc_body)


def _tc2_body(a1c0_ref, a1c1_ref, a2_ref, a3_ref, was_ref, wts_ref, base_ref,
              out_ref):
    a1 = jnp.concatenate([a1c0_ref[...], a1c1_ref[...]], axis=-1)
    out_ref[...] = (
        a1 + base_ref[...]
        + jnp.dot(a2_ref[...], was_ref[...], preferred_element_type=jnp.float32)
        + jnp.dot(a3_ref[...], wts_ref[...], preferred_element_type=jnp.float32)
    )


def _tc2(a1c0, a1c1, a2, a3, was, wts, base):
    return pl.pallas_call(
        _tc2_body,
        grid=(_GRID,),
        in_specs=[
            pl.BlockSpec((_ROW_BLK, HALF), lambda i: (i, 0)),
            pl.BlockSpec((_ROW_BLK, HALF), lambda i: (i, 0)),
            pl.BlockSpec((_ROW_BLK, EDGE_DIM), lambda i: (i, 0)),
            pl.BlockSpec((_ROW_BLK, EDGE_DIM), lambda i: (i, 0)),
            pl.BlockSpec((EDGE_DIM, OUT_DIM), lambda i: (0, 0)),
            pl.BlockSpec((EDGE_DIM, OUT_DIM), lambda i: (0, 0)),
            pl.BlockSpec((_ROW_BLK, OUT_DIM), lambda i: (i, 0)),
        ],
        out_specs=pl.BlockSpec((_ROW_BLK, OUT_DIM), lambda i: (i, 0)),
        out_shape=jax.ShapeDtypeStruct((N_NODES, OUT_DIM), jnp.float32),
    )(a1c0, a1c1, a2, a3, was, wts, base)


def kernel(x, edge_index, edge_attr, edge_t, W_m, b_m, W_r, b_r):
    ei = edge_index.astype(jnp.int32)
    wxs = 0.5 * W_m[:, :NODE_DIM].T
    was = 0.5 * W_m[:, NODE_DIM:NODE_DIM + EDGE_DIM].T
    wts = 0.5 * W_m[:, NODE_DIM + EDGE_DIM:].T
    bias = (b_r + 0.5 * b_m).reshape(1, OUT_DIM)
    z64 = jnp.zeros((CP_ROWS, HALF), jnp.float32)
    z16 = jnp.zeros((CP_ROWS, EDGE_DIM), jnp.float32)

    xw1, xw2, base = _tc1(x, wxs, W_r.T, bias)
    a1c0, a1c1, a2, a3 = _sc_agg(ei, edge_attr, edge_t, xw1, xw2, z64, z16)
    return _tc2(a1c0, a1c1, a2, a3, was, wts, base)
